# baseline (speedup 1.0000x reference)
import functools

import jax
import jax.numpy as jnp
import numpy as np
from jax import lax
from jax.experimental import pallas as pl
from jax.experimental.pallas import tpu as pltpu
from jax.experimental.pallas import tpu_sc as plsc

NC = 2
NS = 16
LANES = 16
NW = NC * NS

MAGIC = 1.5 * 2.0 ** 23


def _lane_reduce(v, op):
    iota = lax.iota(jnp.int32, LANES)
    for sh in (8, 4, 2, 1):
        perm = v.at[jnp.bitwise_xor(iota, sh)].get(mode="promise_in_bounds")
        v = op(v, perm)
    return v


def _transpose16(vs):
    iota = lax.iota(jnp.int32, LANES)
    for k in (1, 2, 4, 8):
        xor = jnp.bitwise_xor(iota, k)
        bit = jnp.bitwise_and(iota, k) != 0
        nvs = list(vs)
        for i in range(LANES):
            if i & k:
                continue
            j = i | k
            a, b = vs[i], vs[j]
            pb = b.at[xor].get(mode="promise_in_bounds")
            pa = a.at[xor].get(mode="promise_in_bounds")
            nvs[i] = jnp.where(bit, pb, a)
            nvs[j] = jnp.where(bit, b, pa)
        vs = nvs
    return vs


def _sc_kernel_body(B, L, D, V, CB, x_hbm, ee1r_hbm, ee2_hbm, out_hbm,
                    idx_v, gidx_v, rows_v, obuf_v, e2_v, tmp_v, loc_v,
                    shared_v, isem0, isem1, gsem0, gsem1, osem0, osem1):
    q2_v = e2_v
    nb_w = B // NW
    nch = nb_w // CB
    VR = V // 2
    rows_per_sid = VR // NS
    RCH = 128
    NSI = L // LANES
    NCJ = D // LANES
    isem = (isem0, isem1)
    gsem = (gsem0, gsem1)
    osem = (osem0, osem1)

    cid = lax.axis_index("c")
    sid = lax.axis_index("s")
    wid = cid * NS + sid

    pinf = jnp.full((LANES,), jnp.inf, jnp.float32)
    ninf = jnp.full((LANES,), -jnp.inf, jnp.float32)
    magic_v = jnp.full((LANES,), MAGIC, jnp.float32)

    pltpu.sync_copy(ee2_hbm, e2_v)

    def red1_chunk(k, carry):
        mn, mx = carry
        pltpu.sync_copy(
            ee1r_hbm.at[pl.ds(sid * rows_per_sid + k * RCH, RCH)],
            rows_v.at[0, pl.ds(0, RCH)])

        def red1_row(r, c2):
            m0, m1 = c2
            for s in range(8):
                v = rows_v[0, r, pl.ds(s * LANES, LANES)]
                m0 = jnp.minimum(m0, v)
                m1 = jnp.maximum(m1, v)
            return (m0, m1)

        return lax.fori_loop(0, RCH, red1_row, (mn, mx))

    mn1, mx1 = lax.fori_loop(0, rows_per_sid // RCH, red1_chunk, (pinf, ninf))

    def red2_row(r, c2):
        m0, m1 = c2
        for s in range(D // LANES):
            v = e2_v[r, pl.ds(s * LANES, LANES)]
            m0 = jnp.minimum(m0, v)
            m1 = jnp.maximum(m1, v)
        return (m0, m1)

    mn2, mx2 = lax.fori_loop(0, L, red2_row, (pinf, ninf))

    tmp_v[0, pl.ds(0, LANES)] = mn1
    tmp_v[0, pl.ds(LANES, LANES)] = mx1
    pltpu.sync_copy(tmp_v, shared_v.at[pl.ds(sid, 1)])
    plsc.subcore_barrier()
    pltpu.sync_copy(shared_v, loc_v)
    accn, accx = pinf, ninf
    for w in range(NS):
        accn = jnp.minimum(accn, loc_v[w, pl.ds(0, LANES)])
        accx = jnp.maximum(accx, loc_v[w, pl.ds(LANES, LANES)])
    minw_v = _lane_reduce(accn, jnp.minimum) + _lane_reduce(mn2, jnp.minimum)
    maxw_v = _lane_reduce(accx, jnp.maximum) + _lane_reduce(mx2, jnp.maximum)

    rng = maxw_v - minw_v
    a_v = 256.0 / rng
    b_v = (-256.0 * minw_v) / rng - 128.0

    def q2_body(l, _):
        for s in range(D // LANES):
            q2_v[l, pl.ds(s * LANES, LANES)] = (
                e2_v[l, pl.ds(s * LANES, LANES)] * a_v + b_v)
        return 0

    lax.fori_loop(0, L, q2_body, 0)

    def b_of(ch):
        return wid * nb_w + ch * CB

    def idx_copy(ch, p):
        return pltpu.make_async_copy(
            x_hbm.at[pl.ds(b_of(ch), CB)], idx_v.at[p], isem[p])

    def gather_copy(ch, p, bb):
        return pltpu.make_async_copy(
            ee1r_hbm.at[gidx_v.at[p, bb]],
            rows_v.at[p, pl.ds(bb * L, L)], gsem[p])

    def out_copy(ch, p):
        return pltpu.make_async_copy(
            obuf_v.at[p], out_hbm.at[pl.ds(b_of(ch), CB)], osem[p])

    def make_gidx(p):
        for bb in range(CB):
            for s in range(NSI):
                v = idx_v[p, bb, pl.ds(s * LANES, LANES)]
                gidx_v[p, bb, pl.ds(s * LANES, LANES)] = (
                    lax.shift_right_logical(v, 1))

    def compute(ch, p):
        @plsc.parallel_loop(0, CB)
        def bb_body(bb):
            base = bb * L
            for si in range(NSI):
                xvec = idx_v[p, bb, pl.ds(si * LANES, LANES)]
                par = lax.shift_left(lax.bitwise_and(xvec, 1), 6)
                offs = [par[i] for i in range(LANES)]

                @plsc.parallel_loop(0, NCJ, unroll=4)
                def cj_body(cj):
                    c0 = cj * LANES
                    vs = []
                    for i in range(LANES):
                        g = rows_v[p, base + si * LANES + i,
                                   pl.ds(offs[i] + c0, LANES)]
                        q2 = q2_v[si * LANES + i, pl.ds(c0, LANES)]
                        z = g * a_v + q2
                        zr = (z + magic_v) - magic_v
                        vs.append(jnp.minimum(zr, 127.0))
                    vs2 = _transpose16(vs)
                    for k in range(LANES):
                        obuf_v[p, bb, c0 + k,
                               pl.ds(si * LANES, LANES)] = vs2[k]

    idx_copy(0, 0).start()
    idx_copy(0, 0).wait()
    make_gidx(0)
    for bb in range(CB):
        gather_copy(0, 0, bb).start()
    idx_copy(1, 1).start()

    def pipe_step(ch, p, q):
        @pl.when(ch + 1 < nch)
        def _():
            idx_copy(ch + 1, q).wait()
            make_gidx(q)
            for bb in range(CB):
                gather_copy(ch + 1, q, bb).start()

        @pl.when(ch >= 2)
        def _():
            out_copy(ch - 2, p).wait()

        for bb in range(CB):
            gather_copy(ch, p, bb).wait()
        compute(ch, p)
        out_copy(ch, p).start()

        @pl.when(ch + 2 < nch)
        def _():
            idx_copy(ch + 2, p).start()

    def pair_body(g, _):
        ch0 = g * 2
        pipe_step(ch0, 0, 1)
        pipe_step(ch0 + 1, 1, 0)
        return 0

    lax.fori_loop(0, nch // 2, pair_body, 0)
    out_copy(nch - 2, 0).wait()
    out_copy(nch - 1, 1).wait()


def kernel(x, sp1, sp2, sb, ee1_weight, ee2_weight):
    B, L = x.shape
    V, D = ee1_weight.shape

    pp_row = np.arange(L) + sp1 + (sp2 - sp1 - L) + (sb - B)
    if not np.array_equal(pp_row, np.arange(L)):
        ee2p = ee2_weight[jnp.asarray(pp_row)]
    else:
        ee2p = ee2_weight

    xi = x.astype(jnp.int32)
    ee1r = ee1_weight.reshape(V // 2, D * 2)

    CB = 4
    mesh = plsc.VectorSubcoreMesh(
        core_axis_name="c", subcore_axis_name="s",
        num_cores=NC, num_subcores=NS)

    body = functools.partial(_sc_kernel_body, B, L, D, V, CB)
    f = pl.kernel(
        body,
        out_type=jax.ShapeDtypeStruct((B, D, L), jnp.float32),
        mesh=mesh,
        scratch_types=[
            pltpu.VMEM((2, CB, L), jnp.int32),
            pltpu.VMEM((2, CB, L), jnp.int32),
            pltpu.VMEM((2, CB * L, 2 * D), jnp.float32),
            pltpu.VMEM((2, CB, D, L), jnp.float32),
            pltpu.VMEM((L, D), jnp.float32),
            pltpu.VMEM((1, 2 * LANES), jnp.float32),
            pltpu.VMEM((NS, 2 * LANES), jnp.float32),
            pltpu.VMEM_SHARED((NS, 2 * LANES), jnp.float32),
            pltpu.SemaphoreType.DMA,
            pltpu.SemaphoreType.DMA,
            pltpu.SemaphoreType.DMA,
            pltpu.SemaphoreType.DMA,
            pltpu.SemaphoreType.DMA,
            pltpu.SemaphoreType.DMA,
        ],
    )
    return f(xi, ee1r, ee2p)


# speedup vs baseline: 1.0277x; 1.0277x over previous
import functools

import jax
import jax.numpy as jnp
import numpy as np
from jax import lax
from jax.experimental import pallas as pl
from jax.experimental.pallas import tpu as pltpu
from jax.experimental.pallas import tpu_sc as plsc

NC = 2
NS = 16
LANES = 16
NW = NC * NS

MAGIC = 1.5 * 2.0 ** 23


def _lane_reduce(v, op):
    iota = lax.iota(jnp.int32, LANES)
    for sh in (8, 4, 2, 1):
        perm = v.at[jnp.bitwise_xor(iota, sh)].get(mode="promise_in_bounds")
        v = op(v, perm)
    return v


def _transpose16(vs):
    iota = lax.iota(jnp.int32, LANES)
    for k in (1, 2, 4, 8):
        xor = jnp.bitwise_xor(iota, k)
        bit = jnp.bitwise_and(iota, k) != 0
        nvs = list(vs)
        for i in range(LANES):
            if i & k:
                continue
            j = i | k
            a, b = vs[i], vs[j]
            pb = b.at[xor].get(mode="promise_in_bounds")
            pa = a.at[xor].get(mode="promise_in_bounds")
            nvs[i] = jnp.where(bit, pb, a)
            nvs[j] = jnp.where(bit, b, pa)
        vs = nvs
    return vs


def _sc_kernel_body(B, L, D, V, CB, x_hbm, ee1r_hbm, ee2_hbm, out_hbm,
                    idx_v, gidx_v, rows_v, obuf_v, e2_v, tmp_v, loc_v,
                    shared_v, isem0, isem1, gsem0, gsem1, osem0, osem1):
    q2_v = e2_v
    nb_w = B // NW
    nch = nb_w // CB
    VR = V // 2
    rows_per_sid = VR // NS
    RCH = 128
    NSI = L // LANES
    NCJ = D // LANES
    isem = (isem0, isem1)
    gsem = (gsem0, gsem1)
    osem = (osem0, osem1)

    cid = lax.axis_index("c")
    sid = lax.axis_index("s")
    wid = cid * NS + sid

    pinf = jnp.full((LANES,), jnp.inf, jnp.float32)
    ninf = jnp.full((LANES,), -jnp.inf, jnp.float32)
    magic_v = jnp.full((LANES,), MAGIC, jnp.float32)

    pltpu.sync_copy(ee2_hbm, e2_v)

    def red1_chunk(k, carry):
        mn, mx = carry
        pltpu.sync_copy(
            ee1r_hbm.at[pl.ds(sid * rows_per_sid + k * RCH, RCH)],
            rows_v.at[0, pl.ds(0, RCH)])

        def red1_row(r, c2):
            m0, m1 = c2
            for s in range(8):
                v = rows_v[0, r, pl.ds(s * LANES, LANES)]
                m0 = jnp.minimum(m0, v)
                m1 = jnp.maximum(m1, v)
            return (m0, m1)

        return lax.fori_loop(0, RCH, red1_row, (mn, mx))

    mn1, mx1 = lax.fori_loop(0, rows_per_sid // RCH, red1_chunk, (pinf, ninf))

    def red2_row(r, c2):
        m0, m1 = c2
        for s in range(D // LANES):
            v = e2_v[r, pl.ds(s * LANES, LANES)]
            m0 = jnp.minimum(m0, v)
            m1 = jnp.maximum(m1, v)
        return (m0, m1)

    mn2, mx2 = lax.fori_loop(0, L, red2_row, (pinf, ninf))

    tmp_v[0, pl.ds(0, LANES)] = mn1
    tmp_v[0, pl.ds(LANES, LANES)] = mx1
    pltpu.sync_copy(tmp_v, shared_v.at[pl.ds(sid, 1)])
    plsc.subcore_barrier()
    pltpu.sync_copy(shared_v, loc_v)
    accn, accx = pinf, ninf
    for w in range(NS):
        accn = jnp.minimum(accn, loc_v[w, pl.ds(0, LANES)])
        accx = jnp.maximum(accx, loc_v[w, pl.ds(LANES, LANES)])
    minw_v = _lane_reduce(accn, jnp.minimum) + _lane_reduce(mn2, jnp.minimum)
    maxw_v = _lane_reduce(accx, jnp.maximum) + _lane_reduce(mx2, jnp.maximum)

    rng = maxw_v - minw_v
    a_v = 256.0 / rng
    b_v = (-256.0 * minw_v) / rng - 128.0

    def q2_body(l, _):
        for s in range(D // LANES):
            q2_v[l, pl.ds(s * LANES, LANES)] = (
                e2_v[l, pl.ds(s * LANES, LANES)] * a_v + b_v)
        return 0

    lax.fori_loop(0, L, q2_body, 0)

    def b_of(ch):
        return wid * nb_w + ch * CB

    def idx_copy(ch, p):
        return pltpu.make_async_copy(
            x_hbm.at[pl.ds(b_of(ch), CB)], idx_v.at[p], isem[p])

    def gather_copy(ch, p, bb):
        return pltpu.make_async_copy(
            ee1r_hbm.at[gidx_v.at[p, bb]],
            rows_v.at[p, pl.ds(bb * L, L)], gsem[p])

    def out_copy(ch, p):
        return pltpu.make_async_copy(
            obuf_v.at[p], out_hbm.at[pl.ds(b_of(ch), CB)], osem[p])

    def make_gidx(p):
        for bb in range(CB):
            for s in range(NSI):
                v = idx_v[p, bb, pl.ds(s * LANES, LANES)]
                gidx_v[p, bb, pl.ds(s * LANES, LANES)] = (
                    lax.shift_right_logical(v, 1))

    def compute(ch, p):
        def bb_body(bb, _):
            base = bb * L
            for si in range(NSI):
                xvec = idx_v[p, bb, pl.ds(si * LANES, LANES)]
                par = lax.shift_left(lax.bitwise_and(xvec, 1), 6)
                offs = [par[i] for i in range(LANES)]

                @plsc.parallel_loop(0, NCJ, unroll=2)
                def cj_body(cj):
                    c0 = cj * LANES
                    vs = []
                    for i in range(LANES):
                        g = rows_v[p, base + si * LANES + i,
                                   pl.ds(offs[i] + c0, LANES)]
                        q2 = q2_v[si * LANES + i, pl.ds(c0, LANES)]
                        z = g * a_v + q2
                        zr = (z + magic_v) - magic_v
                        vs.append(jnp.minimum(zr, 127.0))
                    vs2 = _transpose16(vs)
                    for k in range(LANES):
                        obuf_v[p, bb, c0 + k,
                               pl.ds(si * LANES, LANES)] = vs2[k]
            return 0

        lax.fori_loop(0, CB, bb_body, 0)

    idx_copy(0, 0).start()
    idx_copy(0, 0).wait()
    make_gidx(0)
    for bb in range(CB):
        gather_copy(0, 0, bb).start()
    idx_copy(1, 1).start()

    def pipe_step(ch, p, q):
        @pl.when(ch + 1 < nch)
        def _():
            idx_copy(ch + 1, q).wait()
            make_gidx(q)
            for bb in range(CB):
                gather_copy(ch + 1, q, bb).start()

        @pl.when(ch >= 2)
        def _():
            out_copy(ch - 2, p).wait()

        for bb in range(CB):
            gather_copy(ch, p, bb).wait()
        compute(ch, p)
        out_copy(ch, p).start()

        @pl.when(ch + 2 < nch)
        def _():
            idx_copy(ch + 2, p).start()

    def pair_body(g, _):
        ch0 = g * 2
        pipe_step(ch0, 0, 1)
        pipe_step(ch0 + 1, 1, 0)
        return 0

    lax.fori_loop(0, nch // 2, pair_body, 0)
    out_copy(nch - 2, 0).wait()
    out_copy(nch - 1, 1).wait()


def kernel(x, sp1, sp2, sb, ee1_weight, ee2_weight):
    B, L = x.shape
    V, D = ee1_weight.shape

    pp_row = np.arange(L) + sp1 + (sp2 - sp1 - L) + (sb - B)
    if not np.array_equal(pp_row, np.arange(L)):
        ee2p = ee2_weight[jnp.asarray(pp_row)]
    else:
        ee2p = ee2_weight

    xi = x.astype(jnp.int32)
    ee1r = ee1_weight.reshape(V // 2, D * 2)

    CB = 4
    mesh = plsc.VectorSubcoreMesh(
        core_axis_name="c", subcore_axis_name="s",
        num_cores=NC, num_subcores=NS)

    body = functools.partial(_sc_kernel_body, B, L, D, V, CB)
    f = pl.kernel(
        body,
        out_type=jax.ShapeDtypeStruct((B, D, L), jnp.float32),
        mesh=mesh,
        scratch_types=[
            pltpu.VMEM((2, CB, L), jnp.int32),
            pltpu.VMEM((2, CB, L), jnp.int32),
            pltpu.VMEM((2, CB * L, 2 * D), jnp.float32),
            pltpu.VMEM((2, CB, D, L), jnp.float32),
            pltpu.VMEM((L, D), jnp.float32),
            pltpu.VMEM((1, 2 * LANES), jnp.float32),
            pltpu.VMEM((NS, 2 * LANES), jnp.float32),
            pltpu.VMEM_SHARED((NS, 2 * LANES), jnp.float32),
            pltpu.SemaphoreType.DMA,
            pltpu.SemaphoreType.DMA,
            pltpu.SemaphoreType.DMA,
            pltpu.SemaphoreType.DMA,
            pltpu.SemaphoreType.DMA,
            pltpu.SemaphoreType.DMA,
        ],
    )
    return f(xi, ee1r, ee2p)


# speedup vs baseline: 1.0351x; 1.0072x over previous
import functools

import jax
import jax.numpy as jnp
import numpy as np
from jax import lax
from jax.experimental import pallas as pl
from jax.experimental.pallas import tpu as pltpu
from jax.experimental.pallas import tpu_sc as plsc

NC = 2
NS = 16
LANES = 16
NW = NC * NS

MAGIC = 1.5 * 2.0 ** 23


def _lane_reduce(v, op):
    iota = lax.iota(jnp.int32, LANES)
    for sh in (8, 4, 2, 1):
        perm = v.at[jnp.bitwise_xor(iota, sh)].get(mode="promise_in_bounds")
        v = op(v, perm)
    return v


def _transpose16(vs):
    iota = lax.iota(jnp.int32, LANES)
    for k in (1, 2, 4, 8):
        xor = jnp.bitwise_xor(iota, k)
        bit = jnp.bitwise_and(iota, k) != 0
        nvs = list(vs)
        for i in range(LANES):
            if i & k:
                continue
            j = i | k
            a, b = vs[i], vs[j]
            pb = b.at[xor].get(mode="promise_in_bounds")
            pa = a.at[xor].get(mode="promise_in_bounds")
            nvs[i] = jnp.where(bit, pb, a)
            nvs[j] = jnp.where(bit, b, pa)
        vs = nvs
    return vs


def _sc_kernel_body(B, L, D, V, CB, x_hbm, ee1r_hbm, ee2_hbm, out_hbm,
                    idx_v, gidx_v, rows_v, obuf_v, e2_v, tmp_v, loc_v,
                    shared_v, isem0, isem1, gsem0, gsem1, osem0, osem1):
    q2_v = e2_v
    nb_w = B // NW
    nch = nb_w // CB
    VR = V // 2
    rows_per_sid = VR // NS
    RCH = 128
    NSI = L // LANES
    NCJ = D // LANES
    isem = (isem0, isem1)
    gsem = (gsem0, gsem1)
    osem = (osem0, osem1)

    cid = lax.axis_index("c")
    sid = lax.axis_index("s")
    wid = cid * NS + sid

    pinf = jnp.full((LANES,), jnp.inf, jnp.float32)
    ninf = jnp.full((LANES,), -jnp.inf, jnp.float32)
    magic_v = jnp.full((LANES,), MAGIC, jnp.float32)

    def b_of(ch):
        return wid * nb_w + ch * CB

    def idx_copy(ch, p):
        return pltpu.make_async_copy(
            x_hbm.at[pl.ds(b_of(ch), CB)], idx_v.at[p], isem[p])

    idx_copy(0, 0).start()
    idx_copy(1, 1).start()

    def stage_red(k, slot):
        return pltpu.make_async_copy(
            ee1r_hbm.at[pl.ds(sid * rows_per_sid + k * RCH, RCH)],
            rows_v.at[slot, pl.ds(0, RCH)], gsem[slot])

    NRC = rows_per_sid // RCH
    stage_red(0, 0).start()
    pltpu.sync_copy(ee2_hbm, e2_v)
    mn1, mx1 = pinf, ninf
    for k in range(NRC):
        sl = k & 1
        stage_red(k, sl).wait()
        if k + 1 < NRC:
            stage_red(k + 1, 1 - sl).start()

        def red1_row(r, c2, _sl=sl):
            m0, m1 = c2
            for s in range(8):
                v = rows_v[_sl, r, pl.ds(s * LANES, LANES)]
                m0 = jnp.minimum(m0, v)
                m1 = jnp.maximum(m1, v)
            return (m0, m1)

        mn1, mx1 = lax.fori_loop(0, RCH, red1_row, (mn1, mx1))

    def red2_row(r, c2):
        m0, m1 = c2
        for s in range(D // LANES):
            v = e2_v[r, pl.ds(s * LANES, LANES)]
            m0 = jnp.minimum(m0, v)
            m1 = jnp.maximum(m1, v)
        return (m0, m1)

    mn2, mx2 = lax.fori_loop(0, L, red2_row, (pinf, ninf))

    tmp_v[0, pl.ds(0, LANES)] = mn1
    tmp_v[0, pl.ds(LANES, LANES)] = mx1
    pltpu.sync_copy(tmp_v, shared_v.at[pl.ds(sid, 1)])
    plsc.subcore_barrier()
    pltpu.sync_copy(shared_v, loc_v)
    accn, accx = pinf, ninf
    for w in range(NS):
        accn = jnp.minimum(accn, loc_v[w, pl.ds(0, LANES)])
        accx = jnp.maximum(accx, loc_v[w, pl.ds(LANES, LANES)])
    minw_v = _lane_reduce(accn, jnp.minimum) + _lane_reduce(mn2, jnp.minimum)
    maxw_v = _lane_reduce(accx, jnp.maximum) + _lane_reduce(mx2, jnp.maximum)

    rng = maxw_v - minw_v
    a_v = 256.0 / rng
    b_v = (-256.0 * minw_v) / rng - 128.0

    def q2_body(l, _):
        for s in range(D // LANES):
            q2_v[l, pl.ds(s * LANES, LANES)] = (
                e2_v[l, pl.ds(s * LANES, LANES)] * a_v + b_v)
        return 0

    lax.fori_loop(0, L, q2_body, 0)

    def gather_copy(ch, p, bb):
        return pltpu.make_async_copy(
            ee1r_hbm.at[gidx_v.at[p, bb]],
            rows_v.at[p, pl.ds(bb * L, L)], gsem[p])

    def out_copy(ch, p):
        return pltpu.make_async_copy(
            obuf_v.at[p], out_hbm.at[pl.ds(b_of(ch), CB)], osem[p])

    def make_gidx(p):
        for bb in range(CB):
            for s in range(NSI):
                v = idx_v[p, bb, pl.ds(s * LANES, LANES)]
                gidx_v[p, bb, pl.ds(s * LANES, LANES)] = (
                    lax.shift_right_logical(v, 1))

    def compute(ch, p):
        def bb_body(bb, _):
            base = bb * L
            for si in range(NSI):
                xvec = idx_v[p, bb, pl.ds(si * LANES, LANES)]
                par = lax.shift_left(lax.bitwise_and(xvec, 1), 6)
                offs = [par[i] for i in range(LANES)]

                @plsc.parallel_loop(0, NCJ, unroll=2)
                def cj_body(cj):
                    c0 = cj * LANES
                    vs = []
                    for i in range(LANES):
                        g = rows_v[p, base + si * LANES + i,
                                   pl.ds(offs[i] + c0, LANES)]
                        q2 = q2_v[si * LANES + i, pl.ds(c0, LANES)]
                        z = g * a_v + q2
                        zr = (z + magic_v) - magic_v
                        vs.append(jnp.minimum(zr, 127.0))
                    vs2 = _transpose16(vs)
                    for k in range(LANES):
                        obuf_v[p, bb, c0 + k,
                               pl.ds(si * LANES, LANES)] = vs2[k]
            return 0

        lax.fori_loop(0, CB, bb_body, 0)

    idx_copy(0, 0).wait()
    make_gidx(0)
    for bb in range(CB):
        gather_copy(0, 0, bb).start()

    def pipe_step(ch, p, q):
        @pl.when(ch + 1 < nch)
        def _():
            idx_copy(ch + 1, q).wait()
            make_gidx(q)
            for bb in range(CB):
                gather_copy(ch + 1, q, bb).start()

        @pl.when(ch >= 2)
        def _():
            out_copy(ch - 2, p).wait()

        for bb in range(CB):
            gather_copy(ch, p, bb).wait()
        compute(ch, p)
        out_copy(ch, p).start()

        @pl.when(ch + 2 < nch)
        def _():
            idx_copy(ch + 2, p).start()

    def pair_body(g, _):
        ch0 = g * 2
        pipe_step(ch0, 0, 1)
        pipe_step(ch0 + 1, 1, 0)
        return 0

    lax.fori_loop(0, nch // 2, pair_body, 0)
    out_copy(nch - 2, 0).wait()
    out_copy(nch - 1, 1).wait()


def kernel(x, sp1, sp2, sb, ee1_weight, ee2_weight):
    B, L = x.shape
    V, D = ee1_weight.shape

    pp_row = np.arange(L) + sp1 + (sp2 - sp1 - L) + (sb - B)
    if not np.array_equal(pp_row, np.arange(L)):
        ee2p = ee2_weight[jnp.asarray(pp_row)]
    else:
        ee2p = ee2_weight

    xi = x.astype(jnp.int32)
    ee1r = ee1_weight.reshape(V // 2, D * 2)

    CB = 4
    mesh = plsc.VectorSubcoreMesh(
        core_axis_name="c", subcore_axis_name="s",
        num_cores=NC, num_subcores=NS)

    body = functools.partial(_sc_kernel_body, B, L, D, V, CB)
    f = pl.kernel(
        body,
        out_type=jax.ShapeDtypeStruct((B, D, L), jnp.float32),
        mesh=mesh,
        scratch_types=[
            pltpu.VMEM((2, CB, L), jnp.int32),
            pltpu.VMEM((2, CB, L), jnp.int32),
            pltpu.VMEM((2, CB * L, 2 * D), jnp.float32),
            pltpu.VMEM((2, CB, D, L), jnp.float32),
            pltpu.VMEM((L, D), jnp.float32),
            pltpu.VMEM((1, 2 * LANES), jnp.float32),
            pltpu.VMEM((NS, 2 * LANES), jnp.float32),
            pltpu.VMEM_SHARED((NS, 2 * LANES), jnp.float32),
            pltpu.SemaphoreType.DMA,
            pltpu.SemaphoreType.DMA,
            pltpu.SemaphoreType.DMA,
            pltpu.SemaphoreType.DMA,
            pltpu.SemaphoreType.DMA,
            pltpu.SemaphoreType.DMA,
        ],
    )
    return f(xi, ee1r, ee2p)
